# baseline (speedup 1.0000x reference)
import functools

import jax
import jax.numpy as jnp
from jax import lax
from jax.experimental import pallas as pl
from jax.experimental.pallas import tpu as pltpu
from jax.experimental.pallas import tpu_sc as plsc

_INFO = plsc.get_sparse_core_info()
_NC, _NS = _INFO.num_cores, _INFO.num_subcores
_NW = _NC * _NS

_NCHUNKS = 4


@jax.jit
def kernel(subset, table):
    B = subset.shape[0]
    V, D = table.shape
    assert B % (8 * _NW) == 0
    b_per_w = B // _NW
    assert b_per_w % _NCHUNKS == 0
    ch = b_per_w // _NCHUNKS

    mesh = plsc.VectorSubcoreMesh(core_axis_name="c", subcore_axis_name="s")

    @functools.partial(
        pl.kernel,
        mesh=mesh,
        out_type=jax.ShapeDtypeStruct((B, D), jnp.float32),
        scratch_types=[
            pltpu.VMEM((b_per_w,), jnp.int32),
            pltpu.VMEM((ch, D), jnp.float32),
            pltpu.VMEM((ch, D), jnp.float32),
            pltpu.SemaphoreType.DMA,
            pltpu.SemaphoreType.DMA,
            pltpu.SemaphoreType.DMA,
            pltpu.SemaphoreType.DMA,
        ],
    )
    def gather_kernel(idx_hbm, table_hbm, out_hbm, idx_v, buf0, buf1,
                      gsem0, gsem1, ssem0, ssem1):
        wid = lax.axis_index("s") * _NC + lax.axis_index("c")
        base = wid * b_per_w
        pltpu.sync_copy(idx_hbm.at[pl.ds(base, b_per_w)], idx_v)

        bufs = (buf0, buf1)
        gsems = (gsem0, gsem1)
        ssems = (ssem0, ssem1)
        g = [None] * _NCHUNKS
        s = [None] * _NCHUNKS
        for i in range(_NCHUNKS):
            b = i & 1
            if i >= 2:
                s[i - 2].wait()
            g[i] = pltpu.async_copy(
                table_hbm.at[idx_v.at[pl.ds(i * ch, ch)]], bufs[b], gsems[b])
            if i >= 1:
                g[i - 1].wait()
                s[i - 1] = pltpu.async_copy(
                    bufs[1 - b], out_hbm.at[pl.ds(base + (i - 1) * ch, ch)],
                    ssems[1 - b])
        g[_NCHUNKS - 1].wait()
        last = _NCHUNKS - 1
        s[last] = pltpu.async_copy(
            bufs[last & 1], out_hbm.at[pl.ds(base + last * ch, ch)],
            ssems[last & 1])
        s[last - 1].wait()
        s[last].wait()

    return gather_kernel(subset.astype(jnp.int32), table)


# speedup vs baseline: 1.0413x; 1.0413x over previous
import functools

import jax
import jax.numpy as jnp
from jax import lax
from jax.experimental import pallas as pl
from jax.experimental.pallas import tpu as pltpu
from jax.experimental.pallas import tpu_sc as plsc

_INFO = plsc.get_sparse_core_info()
_NC, _NS = _INFO.num_cores, _INFO.num_subcores
_NW = _NC * _NS


@jax.jit
def kernel(subset, table):
    B = subset.shape[0]
    V, D = table.shape
    assert B % (8 * _NW) == 0
    b_per_w = B // _NW

    mesh = plsc.VectorSubcoreMesh(core_axis_name="c", subcore_axis_name="s")

    @functools.partial(
        pl.kernel,
        mesh=mesh,
        out_type=jax.ShapeDtypeStruct((B, D), jnp.float32),
        scratch_types=[
            pltpu.VMEM((b_per_w,), jnp.int32),
            pltpu.VMEM((b_per_w, D), jnp.float32),
            pltpu.SemaphoreType.DMA,
            pltpu.SemaphoreType.DMA,
        ],
    )
    def gather_kernel(idx_hbm, table_hbm, out_hbm, idx_v, rows_v, isem, gsem):
        wid = lax.axis_index("s") * _NC + lax.axis_index("c")
        base = wid * b_per_w
        ic = pltpu.async_copy(idx_hbm.at[pl.ds(base, b_per_w)], idx_v, isem)
        ic.wait()
        pltpu.async_copy(table_hbm.at[idx_v], rows_v, gsem).wait()
        pltpu.sync_copy(rows_v, out_hbm.at[pl.ds(base, b_per_w)])

    return gather_kernel(subset.astype(jnp.int32), table)
